# baseline (speedup 1.0000x reference)
import functools
import math

import jax
import jax.numpy as jnp
from jax.experimental import pallas as pl
from jax.experimental.pallas import tpu as pltpu

Q = 8
D = 128
M_KEYS = 8192
U = 128
DELTA = 32
GAMMA_ALPHA = 0.1
B = 1024

BBLK = 256
MBLK = 1024


def _scores_body(x_ref, k_ref, out_ref):
    xb = x_ref[...]
    nrm = jnp.sqrt(jnp.sum(xb * xb, axis=1, keepdims=True))
    xb = xb / jnp.maximum(nrm, 1e-12)
    kb = k_ref[0]
    out_ref[0] = jax.lax.dot_general(
        xb, kb, (((1,), (1,)), ((), ())), preferred_element_type=jnp.float32
    )


def _scores(x, K):
    grid = (Q, B // BBLK, M_KEYS // MBLK)
    return pl.pallas_call(
        _scores_body,
        grid=grid,
        in_specs=[
            pl.BlockSpec((BBLK, D), lambda q, i, j: (i, 0)),
            pl.BlockSpec((1, MBLK, D), lambda q, i, j: (q, j, 0)),
        ],
        out_specs=pl.BlockSpec((1, BBLK, MBLK), lambda q, i, j: (q, i, j)),
        out_shape=jax.ShapeDtypeStruct((Q, B, M_KEYS), jnp.float32),
    )(x, K)


def kernel(x, K, M):
    scores = _scores(x, K)
    responses_bqm = jnp.transpose(scores, (1, 0, 2))
    top_responses, top_indices = jax.lax.top_k(responses_bqm, DELTA)
    top_alpha = jax.nn.softmax((GAMMA_ALPHA / math.sqrt(D)) * top_responses, axis=2)
    q_ids = jnp.arange(Q)[None, :, None]
    top_M = M[q_ids, top_indices]
    return jnp.einsum("bqk,bqku->bqu", top_alpha, top_M)


# speedup vs baseline: 11.7457x; 11.7457x over previous
import functools
import math

import jax
import jax.numpy as jnp
from jax import lax
from jax.experimental import pallas as pl
from jax.experimental.pallas import tpu as pltpu
from jax.experimental.pallas import tpu_sc as plsc

Q = 8
D = 128
M_KEYS = 8192
U = 128
DELTA = 32
GAMMA_ALPHA = 0.1
B = 1024
TAU = GAMMA_ALPHA / math.sqrt(D)

R = Q * B
NBLK = M_KEYS // 16

BBLK = 256
MBLK_A = 2048
MBLK = 1024


def _scores_body(x_ref, k_ref, s_ref, bm_ref):
    xb = x_ref[...]
    nrm = jnp.sqrt(jnp.sum(xb * xb, axis=1, keepdims=True))
    xb = xb / jnp.maximum(nrm, 1e-12)
    s = jax.lax.dot_general(
        xb, k_ref[0], (((1,), (1,)), ((), ())), preferred_element_type=jnp.float32
    )
    s_ref[0] = s
    bm_ref[0] = jnp.max(s.reshape(BBLK, MBLK_A // 16, 16), axis=2)


def _scores(x, K):
    grid = (Q, B // BBLK, M_KEYS // MBLK_A)
    return pl.pallas_call(
        _scores_body,
        grid=grid,
        in_specs=[
            pl.BlockSpec((BBLK, D), lambda q, i, j: (i, 0)),
            pl.BlockSpec((1, MBLK_A, D), lambda q, i, j: (q, j, 0)),
        ],
        out_specs=[
            pl.BlockSpec((1, BBLK, MBLK_A), lambda q, i, j: (q, i, j)),
            pl.BlockSpec((1, BBLK, MBLK_A // 16), lambda q, i, j: (q, i, j)),
        ],
        out_shape=[
            jax.ShapeDtypeStruct((Q, B, M_KEYS), jnp.float32),
            jax.ShapeDtypeStruct((Q, B, NBLK), jnp.float32),
        ],
    )(x, K)


def _rev(v):
    return lax.rev(v, (0,))


def _kv_sort(k, v):
    return plsc.sort_key_val(k, v, descending=True)


def _kv_minmax(ka, va, kb, vb):
    m = ka >= kb
    return (
        jnp.where(m, ka, kb),
        jnp.where(m, va, vb),
        jnp.where(m, kb, ka),
        jnp.where(m, vb, va),
    )


def _pair_to_sorted32(ka, va, kb, vb):
    kp, vp, kq, vq = _kv_minmax(ka, va, _rev(kb), _rev(vb))
    k1, v1 = _kv_sort(kp, vp)
    k2, v2 = _kv_sort(kq, vq)
    return k1, v1, k2, v2


def _merge32_top32(a, b):
    ka1, va1, ka2, va2 = a
    kb1, vb1, kb2, vb2 = b
    l1k, l1v, _, _ = _kv_minmax(ka1, va1, _rev(kb2), _rev(vb2))
    l2k, l2v, _, _ = _kv_minmax(ka2, va2, _rev(kb1), _rev(vb1))
    kp, vp, kq, vq = _kv_minmax(l1k, l1v, l2k, l2v)
    k1, v1 = _kv_sort(kp, vp)
    k2, v2 = _kv_sort(kq, vq)
    return (k1, v1, k2, v2)


def _top32_of_refslab(load_fn, nvec):
    leaves = []
    for v in range(0, nvec, 2):
        ka, va = load_fn(v)
        kb, vb = load_fn(v + 1)
        ka, va = _kv_sort(ka, va)
        kb, vb = _kv_sort(kb, vb)
        leaves.append(_pair_to_sorted32(ka, va, kb, vb))
    while len(leaves) > 1:
        nxt = []
        for i in range(0, len(leaves), 2):
            nxt.append(_merge32_top32(leaves[i], leaves[i + 1]))
        leaves = nxt
    return leaves[0]


def _t32_sc(bmax2d, scores2d):
    info = plsc.get_sparse_core_info()
    nc, ns = info.num_cores, info.num_subcores
    nw = nc * ns
    rows_per_w = R // nw

    @functools.partial(
        pl.kernel,
        out_type=jax.ShapeDtypeStruct((R, 16), jnp.float32),
        mesh=plsc.VectorSubcoreMesh(core_axis_name="c", subcore_axis_name="s"),
        compiler_params=pltpu.CompilerParams(needs_layout_passes=False),
        scratch_types=[
            pltpu.VMEM((M_KEYS,), jnp.float32),
            pltpu.VMEM((NBLK,), jnp.float32),
            pltpu.VMEM((rows_per_w, 16), jnp.float32),
        ],
    )
    def k(bmax_hbm, s_hbm, out_hbm, srow_v, bm_v, o_v):
        wid = lax.axis_index("s") * nc + lax.axis_index("c")
        base = wid * rows_per_w

        def body(i, carry):
            row = base + i
            pltpu.sync_copy(s_hbm.at[row], srow_v)
            pltpu.sync_copy(bmax_hbm.at[row], bm_v)

            lane = lax.iota(jnp.int32, 16)

            def load_bm(v):
                return bm_v[pl.ds(v * 16, 16)], lane + v * 16

            _, v1, _, v2 = _top32_of_refslab(load_bm, NBLK // 16)

            def load_cand(v):
                ids = v1 if v < 16 else v2
                return plsc.load_gather(srow_v, [ids * 16 + (v % 16)]), lane

            _, _, k2, _ = _top32_of_refslab(load_cand, DELTA)
            o_v[i] = k2
            return carry

        lax.fori_loop(0, rows_per_w, body, None)
        pltpu.sync_copy(o_v, out_hbm.at[pl.ds(base, rows_per_w)])

    return k(bmax2d, scores2d)


def _combine_body(s_ref, t_ref, m_ref, o_ref, acc, zacc):
    j = pl.program_id(2)
    nj = pl.num_programs(2)

    @pl.when(j == 0)
    def _init():
        acc[...] = jnp.zeros_like(acc)
        zacc[...] = jnp.zeros_like(zacc)

    s = s_ref[0]
    t = t_ref[0, 0, :]
    alpha = jnp.where(s >= t[:, None], jnp.exp(TAU * s), 0.0)
    acc[...] += jax.lax.dot_general(
        alpha, m_ref[0], (((1,), (0,)), ((), ())), preferred_element_type=jnp.float32
    )
    zacc[...] += jnp.sum(alpha.reshape(BBLK, MBLK // 128, 128), axis=1)

    @pl.when(j == nj - 1)
    def _fin():
        z = jnp.sum(zacc[...], axis=1, keepdims=True)
        o_ref[...] = acc[...] / z


def _combine(scores, t32, M):
    grid = (Q, B // BBLK, M_KEYS // MBLK)
    return pl.pallas_call(
        _combine_body,
        grid=grid,
        in_specs=[
            pl.BlockSpec((1, BBLK, MBLK), lambda q, i, j: (q, i, j)),
            pl.BlockSpec((1, 1, BBLK), lambda q, i, j: (q, 0, i)),
            pl.BlockSpec((1, MBLK, U), lambda q, i, j: (q, j, 0)),
        ],
        out_specs=pl.BlockSpec((BBLK, U), lambda q, i, j: (i, q)),
        out_shape=jax.ShapeDtypeStruct((B, Q * U), jnp.float32),
        scratch_shapes=[
            pltpu.VMEM((BBLK, U), jnp.float32),
            pltpu.VMEM((BBLK, 128), jnp.float32),
        ],
    )(scores, t32, M)


def kernel(x, K, M):
    scores, bmax = _scores(x, K)
    top1632 = _t32_sc(bmax.reshape(R, NBLK), scores.reshape(R, M_KEYS))
    t32 = top1632[:, 15].reshape(Q, 1, B)
    out = _combine(scores, t32, M)
    return out.reshape(B, Q, U)


# speedup vs baseline: 21.7850x; 1.8547x over previous
import functools
import math

import jax
import jax.numpy as jnp
from jax import lax
from jax.experimental import pallas as pl
from jax.experimental.pallas import tpu as pltpu
from jax.experimental.pallas import tpu_sc as plsc

Q = 8
D = 128
M_KEYS = 8192
U = 128
DELTA = 32
GAMMA_ALPHA = 0.1
B = 1024
TAU = GAMMA_ALPHA / math.sqrt(D)

R = Q * B
NBLK = M_KEYS // 16

BBLK = 256
MBLK_A = 2048
MBLK = 1024


def _scores_body(x_ref, k_ref, s_ref, bm_ref):
    xb = x_ref[...]
    nrm = jnp.sqrt(jnp.sum(xb * xb, axis=1, keepdims=True))
    xb = xb / jnp.maximum(nrm, 1e-12)
    s = jax.lax.dot_general(
        xb, k_ref[0], (((1,), (1,)), ((), ())), preferred_element_type=jnp.float32
    )
    s_ref[0] = s
    bm_ref[0] = jnp.max(s.reshape(BBLK, 16, MBLK_A // 16), axis=1)


def _scores(x, K):
    grid = (Q, B // BBLK, M_KEYS // MBLK_A)
    return pl.pallas_call(
        _scores_body,
        grid=grid,
        in_specs=[
            pl.BlockSpec((BBLK, D), lambda q, i, j: (i, 0)),
            pl.BlockSpec((1, MBLK_A, D), lambda q, i, j: (q, j, 0)),
        ],
        out_specs=[
            pl.BlockSpec((1, BBLK, MBLK_A), lambda q, i, j: (q, i, j)),
            pl.BlockSpec((1, BBLK, MBLK_A // 16), lambda q, i, j: (q, i, j)),
        ],
        out_shape=[
            jax.ShapeDtypeStruct((Q, B, M_KEYS), jnp.float32),
            jax.ShapeDtypeStruct((Q, B, NBLK), jnp.float32),
        ],
    )(x, K)


def _rev(v):
    return lax.rev(v, (0,))


def _kv_sort(k, v):
    return plsc.sort_key_val(k, v, descending=True)


def _kv_minmax(ka, va, kb, vb):
    m = ka >= kb
    return (
        jnp.where(m, ka, kb),
        jnp.where(m, va, vb),
        jnp.where(m, kb, ka),
        jnp.where(m, vb, va),
    )


def _pair_to_sorted32(ka, va, kb, vb):
    kp, vp, kq, vq = _kv_minmax(ka, va, _rev(kb), _rev(vb))
    k1, v1 = _kv_sort(kp, vp)
    k2, v2 = _kv_sort(kq, vq)
    return k1, v1, k2, v2


def _merge32_top32(a, b):
    ka1, va1, ka2, va2 = a
    kb1, vb1, kb2, vb2 = b
    l1k, l1v, _, _ = _kv_minmax(ka1, va1, _rev(kb2), _rev(vb2))
    l2k, l2v, _, _ = _kv_minmax(ka2, va2, _rev(kb1), _rev(vb1))
    kp, vp, kq, vq = _kv_minmax(l1k, l1v, l2k, l2v)
    k1, v1 = _kv_sort(kp, vp)
    k2, v2 = _kv_sort(kq, vq)
    return (k1, v1, k2, v2)


def _top32_of_refslab(load_fn, nvec):
    leaves = []
    for v in range(0, nvec, 2):
        ka, va = load_fn(v)
        kb, vb = load_fn(v + 1)
        ka, va = _kv_sort(ka, va)
        kb, vb = _kv_sort(kb, vb)
        leaves.append(_pair_to_sorted32(ka, va, kb, vb))
    while len(leaves) > 1:
        nxt = []
        for i in range(0, len(leaves), 2):
            nxt.append(_merge32_top32(leaves[i], leaves[i + 1]))
        leaves = nxt
    return leaves[0]


def _t32_sc(bmax2d, scores2d):
    info = plsc.get_sparse_core_info()
    nc, ns = info.num_cores, info.num_subcores
    nw = nc * ns
    rows_per_w = R // nw

    @functools.partial(
        pl.kernel,
        out_type=jax.ShapeDtypeStruct((R, 16), jnp.float32),
        mesh=plsc.VectorSubcoreMesh(core_axis_name="c", subcore_axis_name="s"),
        compiler_params=pltpu.CompilerParams(needs_layout_passes=False),
        scratch_types=[
            pltpu.VMEM((M_KEYS,), jnp.float32),
            pltpu.VMEM((NBLK,), jnp.float32),
            pltpu.VMEM((rows_per_w, 16), jnp.float32),
        ],
    )
    def k(bmax_hbm, s_hbm, out_hbm, srow_v, bm_v, o_v):
        wid = lax.axis_index("s") * nc + lax.axis_index("c")
        base = wid * rows_per_w

        def body(i, carry):
            row = base + i
            pltpu.sync_copy(s_hbm.at[row], srow_v)
            pltpu.sync_copy(bmax_hbm.at[row], bm_v)

            lane = lax.iota(jnp.int32, 16)

            def load_bm(v):
                return bm_v[pl.ds(v * 16, 16)], lane + v * 16

            _, v1, _, v2 = _top32_of_refslab(load_bm, NBLK // 16)

            base1 = ((v1 & -128) << 4) | (v1 & 127)
            base2 = ((v2 & -128) << 4) | (v2 & 127)

            def load_cand(v):
                base = base1 if v < 16 else base2
                return plsc.load_gather(srow_v, [base + ((v % 16) << 7)]), lane

            _, _, k2, _ = _top32_of_refslab(load_cand, DELTA)
            o_v[i] = k2
            return carry

        lax.fori_loop(0, rows_per_w, body, None)
        pltpu.sync_copy(o_v, out_hbm.at[pl.ds(base, rows_per_w)])

    return k(bmax2d, scores2d)


def _combine_body(s_ref, t_ref, m_ref, o_ref, acc, zacc):
    j = pl.program_id(2)
    nj = pl.num_programs(2)

    @pl.when(j == 0)
    def _init():
        acc[...] = jnp.zeros_like(acc)
        zacc[...] = jnp.zeros_like(zacc)

    s = s_ref[0]
    t = t_ref[0]
    alpha = jnp.where(s >= t, jnp.exp(TAU * s), 0.0)
    acc[...] += jax.lax.dot_general(
        alpha, m_ref[0], (((1,), (0,)), ((), ())), preferred_element_type=jnp.float32
    )
    zacc[...] += jnp.sum(alpha.reshape(BBLK, MBLK // 128, 128), axis=1)

    @pl.when(j == nj - 1)
    def _fin():
        z = jnp.sum(zacc[...], axis=1, keepdims=True)
        o_ref[...] = acc[...] / z


def _combine(scores, t32, M):
    grid = (Q, B // BBLK, M_KEYS // MBLK)
    return pl.pallas_call(
        _combine_body,
        grid=grid,
        in_specs=[
            pl.BlockSpec((1, BBLK, MBLK), lambda q, i, j: (q, i, j)),
            pl.BlockSpec((1, BBLK, 1), lambda q, i, j: (q, i, 0)),
            pl.BlockSpec((1, MBLK, U), lambda q, i, j: (q, j, 0)),
        ],
        out_specs=pl.BlockSpec((BBLK, U), lambda q, i, j: (i, q)),
        out_shape=jax.ShapeDtypeStruct((B, Q * U), jnp.float32),
        scratch_shapes=[
            pltpu.VMEM((BBLK, U), jnp.float32),
            pltpu.VMEM((BBLK, 128), jnp.float32),
        ],
    )(scores, t32, M)


def kernel(x, K, M):
    scores, bmax = _scores(x, K)
    top1632 = _t32_sc(bmax.reshape(R, NBLK), scores.reshape(R, M_KEYS))
    t32 = top1632[:, 15].reshape(Q, B, 1)
    out = _combine(scores, t32, M)
    return out.reshape(B, Q, U)


# speedup vs baseline: 31.5145x; 1.4466x over previous
import functools
import math

import jax
import jax.numpy as jnp
from jax import lax
from jax.experimental import pallas as pl
from jax.experimental.pallas import tpu as pltpu
from jax.experimental.pallas import tpu_sc as plsc

Q = 8
D = 128
M_KEYS = 8192
U = 128
DELTA = 32
GAMMA_ALPHA = 0.1
B = 1024
TAU = GAMMA_ALPHA / math.sqrt(D)

R = Q * B
NBLK = M_KEYS // 16

BBLK = 256
MBLK_A = 2048
MBLK = 1024


def _scores_body(x_ref, k_ref, s_ref, bm_ref):
    xb = x_ref[...]
    nrm = jnp.sqrt(jnp.sum(xb * xb, axis=1, keepdims=True))
    xb = xb / jnp.maximum(nrm, 1e-12)
    s = jax.lax.dot_general(
        xb, k_ref[0], (((1,), (1,)), ((), ())), preferred_element_type=jnp.float32
    )
    s_ref[0] = s
    bm_ref[0] = jnp.max(s.reshape(BBLK, 16, MBLK_A // 16), axis=1)


def _scores(x, K):
    grid = (Q, B // BBLK, M_KEYS // MBLK_A)
    return pl.pallas_call(
        _scores_body,
        grid=grid,
        in_specs=[
            pl.BlockSpec((BBLK, D), lambda q, i, j: (i, 0)),
            pl.BlockSpec((1, MBLK_A, D), lambda q, i, j: (q, j, 0)),
        ],
        out_specs=[
            pl.BlockSpec((1, BBLK, MBLK_A), lambda q, i, j: (q, i, j)),
            pl.BlockSpec((1, BBLK, MBLK_A // 16), lambda q, i, j: (q, i, j)),
        ],
        out_shape=[
            jax.ShapeDtypeStruct((Q, B, M_KEYS), jnp.float32),
            jax.ShapeDtypeStruct((Q, B, NBLK), jnp.float32),
        ],
    )(x, K)


def _rev(v):
    return lax.rev(v, (0,))


def _kv_sort(k, v):
    return plsc.sort_key_val(k, v, descending=True)


def _kv_minmax(ka, va, kb, vb):
    m = ka >= kb
    return (
        jnp.where(m, ka, kb),
        jnp.where(m, va, vb),
        jnp.where(m, kb, ka),
        jnp.where(m, vb, va),
    )


def _pair_to_sorted32(ka, va, kb, vb):
    kp, vp, kq, vq = _kv_minmax(ka, va, _rev(kb), _rev(vb))
    k1, v1 = _kv_sort(kp, vp)
    k2, v2 = _kv_sort(kq, vq)
    return k1, v1, k2, v2


def _merge32_top32(a, b):
    ka1, va1, ka2, va2 = a
    kb1, vb1, kb2, vb2 = b
    l1k, l1v, _, _ = _kv_minmax(ka1, va1, _rev(kb2), _rev(vb2))
    l2k, l2v, _, _ = _kv_minmax(ka2, va2, _rev(kb1), _rev(vb1))
    kp, vp, kq, vq = _kv_minmax(l1k, l1v, l2k, l2v)
    k1, v1 = _kv_sort(kp, vp)
    k2, v2 = _kv_sort(kq, vq)
    return (k1, v1, k2, v2)


def _top32_of_refslab(load_fn, nvec):
    leaves = []
    for v in range(0, nvec, 2):
        ka, va = load_fn(v)
        kb, vb = load_fn(v + 1)
        ka, va = _kv_sort(ka, va)
        kb, vb = _kv_sort(kb, vb)
        leaves.append(_pair_to_sorted32(ka, va, kb, vb))
    while len(leaves) > 1:
        nxt = []
        for i in range(0, len(leaves), 2):
            nxt.append(_merge32_top32(leaves[i], leaves[i + 1]))
        leaves = nxt
    return leaves[0]


def _k_sort(k):
    return plsc.sort_key_val(k, k, descending=True)[0]


def _keys_pair_to_sorted32(ka, kb):
    kp = jnp.maximum(ka, _rev(kb))
    kq = jnp.minimum(ka, _rev(kb))
    return _k_sort(kp), _k_sort(kq)


def _keys_merge32(a, b):
    ka1, ka2 = a
    kb1, kb2 = b
    l1 = jnp.maximum(ka1, _rev(kb2))
    l2 = jnp.maximum(ka2, _rev(kb1))
    kp = jnp.maximum(l1, l2)
    kq = jnp.minimum(l1, l2)
    return _k_sort(kp), _k_sort(kq)


def _rank32_of_keys(load_fn, nvec):
    leaves = []
    for v in range(0, nvec, 2):
        ka = _k_sort(load_fn(v))
        kb = _k_sort(load_fn(v + 1))
        leaves.append(_keys_pair_to_sorted32(ka, kb))
    while len(leaves) > 1:
        nxt = []
        for i in range(0, len(leaves), 2):
            nxt.append(_keys_merge32(leaves[i], leaves[i + 1]))
        leaves = nxt
    return lax.reduce_min(leaves[0][1], (0,))


def _t32_sc(bmax2d, scores2d):
    info = plsc.get_sparse_core_info()
    nc, ns = info.num_cores, info.num_subcores
    nw = nc * ns
    rows_per_w = R // nw

    @functools.partial(
        pl.kernel,
        out_type=jax.ShapeDtypeStruct((R, 16), jnp.float32),
        mesh=plsc.VectorSubcoreMesh(core_axis_name="c", subcore_axis_name="s"),
        compiler_params=pltpu.CompilerParams(needs_layout_passes=False),
        scratch_types=[
            pltpu.VMEM((M_KEYS,), jnp.float32),
            pltpu.VMEM((M_KEYS,), jnp.float32),
            pltpu.VMEM((NBLK,), jnp.float32),
            pltpu.VMEM((NBLK,), jnp.float32),
            pltpu.VMEM((rows_per_w, 16), jnp.float32),
            pltpu.SemaphoreType.DMA,
            pltpu.SemaphoreType.DMA,
            pltpu.SemaphoreType.DMA,
            pltpu.SemaphoreType.DMA,
        ],
    )
    def k(bmax_hbm, s_hbm, out_hbm, srow0, srow1, bm0, bm1, o_v,
          sem_s0, sem_s1, sem_b0, sem_b1):
        wid = lax.axis_index("s") * nc + lax.axis_index("c")
        base = wid * rows_per_w
        lane = lax.iota(jnp.int32, 16)
        last = rows_per_w - 1
        srows = (srow0, srow1)
        bms = (bm0, bm1)
        sems_s = (sem_s0, sem_s1)
        sems_b = (sem_b0, sem_b1)

        for b in (0, 1):
            pltpu.async_copy(s_hbm.at[base + b], srows[b], sems_s[b])
            pltpu.async_copy(bmax_hbm.at[base + b], bms[b], sems_b[b])

        def process(i, buf):
            srow = srows[buf]
            bm = bms[buf]
            sem_s = sems_s[buf]
            sem_b = sems_b[buf]
            pltpu.make_async_copy(s_hbm.at[base], srow, sem_s).wait()
            pltpu.make_async_copy(bmax_hbm.at[base], bm, sem_b).wait()

            def load_bm(v):
                return bm[pl.ds(v * 16, 16)], lane + v * 16

            _, v1, _, v2 = _top32_of_refslab(load_bm, NBLK // 16)

            base1 = ((v1 & -128) << 4) | (v1 & 127)
            base2 = ((v2 & -128) << 4) | (v2 & 127)

            def load_cand(v):
                vb = base1 if v < 16 else base2
                return plsc.load_gather(srow, [vb + ((v % 16) << 7)])

            t32 = _rank32_of_keys(load_cand, DELTA)
            o_v[i] = jnp.full((16,), t32, jnp.float32)
            nxt = base + jnp.minimum(i + 2, last)
            pltpu.async_copy(s_hbm.at[nxt], srow, sem_s)
            pltpu.async_copy(bmax_hbm.at[nxt], bm, sem_b)

        def body(g, carry):
            process(2 * g, 0)
            process(2 * g + 1, 1)
            return carry

        lax.fori_loop(0, rows_per_w // 2, body, None)
        for b in (0, 1):
            pltpu.make_async_copy(s_hbm.at[base], srows[b], sems_s[b]).wait()
            pltpu.make_async_copy(bmax_hbm.at[base], bms[b], sems_b[b]).wait()
        pltpu.sync_copy(o_v, out_hbm.at[pl.ds(base, rows_per_w)])

    return k(bmax2d, scores2d)


def _combine_body(s_ref, t_ref, m_ref, o_ref, acc, zacc):
    j = pl.program_id(2)
    nj = pl.num_programs(2)

    @pl.when(j == 0)
    def _init():
        acc[...] = jnp.zeros_like(acc)
        zacc[...] = jnp.zeros_like(zacc)

    s = s_ref[0]
    t = t_ref[0]
    alpha = jnp.where(s >= t, jnp.exp(TAU * s), 0.0)
    acc[...] += jax.lax.dot_general(
        alpha, m_ref[0], (((1,), (0,)), ((), ())), preferred_element_type=jnp.float32
    )
    zacc[...] += jnp.sum(alpha.reshape(BBLK, MBLK // 128, 128), axis=1)

    @pl.when(j == nj - 1)
    def _fin():
        z = jnp.sum(zacc[...], axis=1, keepdims=True)
        o_ref[...] = acc[...] / z


def _combine(scores, t32, M):
    grid = (Q, B // BBLK, M_KEYS // MBLK)
    return pl.pallas_call(
        _combine_body,
        grid=grid,
        in_specs=[
            pl.BlockSpec((1, BBLK, MBLK), lambda q, i, j: (q, i, j)),
            pl.BlockSpec((1, BBLK, 1), lambda q, i, j: (q, i, 0)),
            pl.BlockSpec((1, MBLK, U), lambda q, i, j: (q, j, 0)),
        ],
        out_specs=pl.BlockSpec((BBLK, U), lambda q, i, j: (i, q)),
        out_shape=jax.ShapeDtypeStruct((B, Q * U), jnp.float32),
        scratch_shapes=[
            pltpu.VMEM((BBLK, U), jnp.float32),
            pltpu.VMEM((BBLK, 128), jnp.float32),
        ],
    )(scores, t32, M)


def kernel(x, K, M):
    scores, bmax = _scores(x, K)
    top1632 = _t32_sc(bmax.reshape(R, NBLK), scores.reshape(R, M_KEYS))
    t32 = top1632[:, :1].reshape(Q, B, 1)
    out = _combine(scores, t32, M)
    return out.reshape(B, Q, U)


# speedup vs baseline: 35.7496x; 1.1344x over previous
import functools
import math

import jax
import jax.numpy as jnp
from jax import lax
from jax.experimental import pallas as pl
from jax.experimental.pallas import tpu as pltpu
from jax.experimental.pallas import tpu_sc as plsc

Q = 8
D = 128
M_KEYS = 8192
U = 128
DELTA = 32
GAMMA_ALPHA = 0.1
B = 1024
TAU = GAMMA_ALPHA / math.sqrt(D)

R = Q * B
NBLK = M_KEYS // 16

BBLK = 256
MBLK_A = 2048
MBLK = 2048


def _scores_body(x_ref, k_ref, s_ref, bm_ref):
    xb = x_ref[...]
    nrm = jnp.sqrt(jnp.sum(xb * xb, axis=1, keepdims=True))
    xb = xb / jnp.maximum(nrm, 1e-12)
    s = jax.lax.dot_general(
        xb, k_ref[0], (((1,), (1,)), ((), ())), preferred_element_type=jnp.float32
    )
    s_ref[0] = s
    bm_ref[0] = jnp.max(s.reshape(BBLK, 16, MBLK_A // 16), axis=1)


def _scores(x, K):
    grid = (Q, B // BBLK, M_KEYS // MBLK_A)
    return pl.pallas_call(
        _scores_body,
        grid=grid,
        in_specs=[
            pl.BlockSpec((BBLK, D), lambda q, i, j: (i, 0)),
            pl.BlockSpec((1, MBLK_A, D), lambda q, i, j: (q, j, 0)),
        ],
        out_specs=[
            pl.BlockSpec((1, BBLK, MBLK_A), lambda q, i, j: (q, i, j)),
            pl.BlockSpec((1, BBLK, MBLK_A // 16), lambda q, i, j: (q, i, j)),
        ],
        out_shape=[
            jax.ShapeDtypeStruct((Q, B, M_KEYS), jnp.float32),
            jax.ShapeDtypeStruct((Q, B, NBLK), jnp.float32),
        ],
    )(x, K)


def _rev(v):
    return lax.rev(v, (0,))


def _kv_sort(k, v):
    return plsc.sort_key_val(k, v, descending=True)


def _kv_minmax(ka, va, kb, vb):
    m = ka >= kb
    return (
        jnp.where(m, ka, kb),
        jnp.where(m, va, vb),
        jnp.where(m, kb, ka),
        jnp.where(m, vb, va),
    )


def _pair_to_sorted32(ka, va, kb, vb):
    kp, vp, kq, vq = _kv_minmax(ka, va, _rev(kb), _rev(vb))
    k1, v1 = _kv_sort(kp, vp)
    k2, v2 = _kv_sort(kq, vq)
    return k1, v1, k2, v2


def _merge32_top32(a, b):
    ka1, va1, ka2, va2 = a
    kb1, vb1, kb2, vb2 = b
    l1k, l1v, _, _ = _kv_minmax(ka1, va1, _rev(kb2), _rev(vb2))
    l2k, l2v, _, _ = _kv_minmax(ka2, va2, _rev(kb1), _rev(vb1))
    kp, vp, kq, vq = _kv_minmax(l1k, l1v, l2k, l2v)
    k1, v1 = _kv_sort(kp, vp)
    k2, v2 = _kv_sort(kq, vq)
    return (k1, v1, k2, v2)


def _top32_of_refslab(load_fn, nvec):
    leaves = []
    for v in range(0, nvec, 2):
        ka, va = load_fn(v)
        kb, vb = load_fn(v + 1)
        ka, va = _kv_sort(ka, va)
        kb, vb = _kv_sort(kb, vb)
        leaves.append(_pair_to_sorted32(ka, va, kb, vb))
    while len(leaves) > 1:
        nxt = []
        for i in range(0, len(leaves), 2):
            nxt.append(_merge32_top32(leaves[i], leaves[i + 1]))
        leaves = nxt
    return leaves[0]


def _k_sort(k):
    return plsc.sort_key_val(k, k, descending=True)[0]


def _keys_pair_to_sorted32(ka, kb):
    kp = jnp.maximum(ka, _rev(kb))
    kq = jnp.minimum(ka, _rev(kb))
    return _k_sort(kp), _k_sort(kq)


def _keys_merge32(a, b):
    ka1, ka2 = a
    kb1, kb2 = b
    l1 = jnp.maximum(ka1, _rev(kb2))
    l2 = jnp.maximum(ka2, _rev(kb1))
    kp = jnp.maximum(l1, l2)
    kq = jnp.minimum(l1, l2)
    return _k_sort(kp), _k_sort(kq)


def _rank32_of_keys(load_fn, nvec):
    leaves = []
    for v in range(0, nvec, 2):
        ka = _k_sort(load_fn(v))
        kb = _k_sort(load_fn(v + 1))
        leaves.append(_keys_pair_to_sorted32(ka, kb))
    while len(leaves) > 1:
        nxt = []
        for i in range(0, len(leaves), 2):
            nxt.append(_keys_merge32(leaves[i], leaves[i + 1]))
        leaves = nxt
    return lax.reduce_min(leaves[0][1], (0,))


def _t32_sc(bmax2d, scores2d):
    info = plsc.get_sparse_core_info()
    nc, ns = info.num_cores, info.num_subcores
    nw = nc * ns
    rows_per_w = R // nw

    @functools.partial(
        pl.kernel,
        out_type=jax.ShapeDtypeStruct((R, 16), jnp.float32),
        mesh=plsc.VectorSubcoreMesh(core_axis_name="c", subcore_axis_name="s"),
        compiler_params=pltpu.CompilerParams(needs_layout_passes=False),
        scratch_types=[
            pltpu.VMEM((M_KEYS,), jnp.float32),
            pltpu.VMEM((M_KEYS,), jnp.float32),
            pltpu.VMEM((NBLK,), jnp.float32),
            pltpu.VMEM((NBLK,), jnp.float32),
            pltpu.VMEM((rows_per_w, 16), jnp.float32),
            pltpu.SemaphoreType.DMA,
            pltpu.SemaphoreType.DMA,
            pltpu.SemaphoreType.DMA,
            pltpu.SemaphoreType.DMA,
        ],
    )
    def k(bmax_hbm, s_hbm, out_hbm, srow0, srow1, bm0, bm1, o_v,
          sem_s0, sem_s1, sem_b0, sem_b1):
        wid = lax.axis_index("s") * nc + lax.axis_index("c")
        base = wid * rows_per_w
        lane = lax.iota(jnp.int32, 16)
        last = rows_per_w - 1
        srows = (srow0, srow1)
        bms = (bm0, bm1)
        sems_s = (sem_s0, sem_s1)
        sems_b = (sem_b0, sem_b1)

        for b in (0, 1):
            pltpu.async_copy(s_hbm.at[base + b], srows[b], sems_s[b])
            pltpu.async_copy(bmax_hbm.at[base + b], bms[b], sems_b[b])

        def process(i, buf):
            srow = srows[buf]
            bm = bms[buf]
            sem_s = sems_s[buf]
            sem_b = sems_b[buf]
            pltpu.make_async_copy(s_hbm.at[base], srow, sem_s).wait()
            pltpu.make_async_copy(bmax_hbm.at[base], bm, sem_b).wait()

            def load_bm(v):
                return bm[pl.ds(v * 16, 16)], lane + v * 16

            _, v1, _, v2 = _top32_of_refslab(load_bm, NBLK // 16)

            base1 = ((v1 & -128) << 4) | (v1 & 127)
            base2 = ((v2 & -128) << 4) | (v2 & 127)

            def load_cand(v):
                vb = base1 if v < 16 else base2
                return plsc.load_gather(srow, [vb + ((v % 16) << 7)])

            t32 = _rank32_of_keys(load_cand, DELTA)
            o_v[i] = jnp.full((16,), t32, jnp.float32)
            nxt = base + jnp.minimum(i + 2, last)
            pltpu.async_copy(s_hbm.at[nxt], srow, sem_s)
            pltpu.async_copy(bmax_hbm.at[nxt], bm, sem_b)

        def body(g, carry):
            process(2 * g, 0)
            process(2 * g + 1, 1)
            return carry

        lax.fori_loop(0, rows_per_w // 2, body, None)
        for b in (0, 1):
            pltpu.make_async_copy(s_hbm.at[base], srows[b], sems_s[b]).wait()
            pltpu.make_async_copy(bmax_hbm.at[base], bms[b], sems_b[b]).wait()
        pltpu.sync_copy(o_v, out_hbm.at[pl.ds(base, rows_per_w)])

    return k(bmax2d, scores2d)


def _combine_body(s_ref, t_ref, m_ref, o_ref, acc, zacc):
    j = pl.program_id(2)
    nj = pl.num_programs(2)

    @pl.when(j == 0)
    def _init():
        acc[...] = jnp.zeros_like(acc)
        zacc[...] = jnp.zeros_like(zacc)

    s = s_ref[0]
    t = t_ref[0][:, :1]
    alpha = jnp.where(s >= t, jnp.exp(TAU * s), 0.0)
    acc[...] += jax.lax.dot_general(
        alpha, m_ref[0], (((1,), (0,)), ((), ())), preferred_element_type=jnp.float32
    )
    zacc[...] += jnp.sum(alpha.reshape(BBLK, MBLK // 128, 128), axis=1)

    @pl.when(j == nj - 1)
    def _fin():
        z = jnp.sum(zacc[...], axis=1, keepdims=True)
        o_ref[...] = acc[...] / z


def _combine(scores, t32, M):
    grid = (Q, B // BBLK, M_KEYS // MBLK)
    return pl.pallas_call(
        _combine_body,
        grid=grid,
        in_specs=[
            pl.BlockSpec((1, BBLK, MBLK), lambda q, i, j: (q, i, j)),
            pl.BlockSpec((1, BBLK, 16), lambda q, i, j: (q, i, 0)),
            pl.BlockSpec((1, MBLK, U), lambda q, i, j: (q, j, 0)),
        ],
        out_specs=pl.BlockSpec((BBLK, U), lambda q, i, j: (i, q)),
        out_shape=jax.ShapeDtypeStruct((B, Q * U), jnp.float32),
        scratch_shapes=[
            pltpu.VMEM((BBLK, U), jnp.float32),
            pltpu.VMEM((BBLK, 128), jnp.float32),
        ],
    )(scores, t32, M)


def kernel(x, K, M):
    scores, bmax = _scores(x, K)
    top1632 = _t32_sc(bmax.reshape(R, NBLK), scores.reshape(R, M_KEYS))
    out = _combine(scores, top1632.reshape(Q, B, 16), M)
    return out.reshape(B, Q, U)


# speedup vs baseline: 43.5157x; 1.2172x over previous
import functools
import math

import jax
import jax.numpy as jnp
from jax import lax
from jax.experimental import pallas as pl
from jax.experimental.pallas import tpu as pltpu
from jax.experimental.pallas import tpu_sc as plsc

Q = 8
D = 128
M_KEYS = 8192
U = 128
DELTA = 32
GAMMA_ALPHA = 0.1
B = 1024
TAU = GAMMA_ALPHA / math.sqrt(D)

R = Q * B
QH = Q // 2
RH = QH * B
NBLK = M_KEYS // 16

BBLK = 256
MBLK_A = 2048
MBLK = 2048


def _scores_body(x_ref, k_ref, s_ref, bm_ref):
    xb = x_ref[...]
    nrm = jnp.sqrt(jnp.sum(xb * xb, axis=1, keepdims=True))
    xb = xb / jnp.maximum(nrm, 1e-12)
    s = jax.lax.dot_general(
        xb, k_ref[0], (((1,), (1,)), ((), ())), preferred_element_type=jnp.float32
    )
    s_ref[0] = s
    bm_ref[0] = jnp.max(s.reshape(BBLK, 16, MBLK_A // 16), axis=1)


def _scores(x, K):
    grid = (QH, B // BBLK, M_KEYS // MBLK_A)
    return pl.pallas_call(
        _scores_body,
        grid=grid,
        in_specs=[
            pl.BlockSpec((BBLK, D), lambda q, i, j: (i, 0)),
            pl.BlockSpec((1, MBLK_A, D), lambda q, i, j: (q, j, 0)),
        ],
        out_specs=[
            pl.BlockSpec((1, BBLK, MBLK_A), lambda q, i, j: (q, i, j)),
            pl.BlockSpec((1, BBLK, MBLK_A // 16), lambda q, i, j: (q, i, j)),
        ],
        out_shape=[
            jax.ShapeDtypeStruct((QH, B, M_KEYS), jnp.float32),
            jax.ShapeDtypeStruct((QH, B, NBLK), jnp.float32),
        ],
    )(x, K)


def _rev(v):
    return lax.rev(v, (0,))


def _kv_sort(k, v):
    return plsc.sort_key_val(k, v, descending=True)


def _kv_minmax(ka, va, kb, vb):
    m = ka >= kb
    return (
        jnp.where(m, ka, kb),
        jnp.where(m, va, vb),
        jnp.where(m, kb, ka),
        jnp.where(m, vb, va),
    )


def _pair_to_sorted32(ka, va, kb, vb):
    kp, vp, kq, vq = _kv_minmax(ka, va, _rev(kb), _rev(vb))
    k1, v1 = _kv_sort(kp, vp)
    k2, v2 = _kv_sort(kq, vq)
    return k1, v1, k2, v2


def _merge32_top32(a, b):
    ka1, va1, ka2, va2 = a
    kb1, vb1, kb2, vb2 = b
    l1k, l1v, _, _ = _kv_minmax(ka1, va1, _rev(kb2), _rev(vb2))
    l2k, l2v, _, _ = _kv_minmax(ka2, va2, _rev(kb1), _rev(vb1))
    kp, vp, kq, vq = _kv_minmax(l1k, l1v, l2k, l2v)
    k1, v1 = _kv_sort(kp, vp)
    k2, v2 = _kv_sort(kq, vq)
    return (k1, v1, k2, v2)


def _top32_of_refslab(load_fn, nvec):
    leaves = []
    for v in range(0, nvec, 2):
        ka, va = load_fn(v)
        kb, vb = load_fn(v + 1)
        ka, va = _kv_sort(ka, va)
        kb, vb = _kv_sort(kb, vb)
        leaves.append(_pair_to_sorted32(ka, va, kb, vb))
    while len(leaves) > 1:
        nxt = []
        for i in range(0, len(leaves), 2):
            nxt.append(_merge32_top32(leaves[i], leaves[i + 1]))
        leaves = nxt
    return leaves[0]


def _k_sort(k):
    return plsc.sort_key_val(k, k, descending=True)[0]


def _keys_pair_to_sorted32(ka, kb):
    kp = jnp.maximum(ka, _rev(kb))
    kq = jnp.minimum(ka, _rev(kb))
    return _k_sort(kp), _k_sort(kq)


def _keys_merge32(a, b):
    ka1, ka2 = a
    kb1, kb2 = b
    l1 = jnp.maximum(ka1, _rev(kb2))
    l2 = jnp.maximum(ka2, _rev(kb1))
    kp = jnp.maximum(l1, l2)
    kq = jnp.minimum(l1, l2)
    return _k_sort(kp), _k_sort(kq)


def _rank32_of_keys(load_fn, nvec):
    leaves = []
    for v in range(0, nvec, 2):
        ka = _k_sort(load_fn(v))
        kb = _k_sort(load_fn(v + 1))
        leaves.append(_keys_pair_to_sorted32(ka, kb))
    while len(leaves) > 1:
        nxt = []
        for i in range(0, len(leaves), 2):
            nxt.append(_keys_merge32(leaves[i], leaves[i + 1]))
        leaves = nxt
    return lax.reduce_min(leaves[0][1], (0,))


def _t32_sc(bmax2d, scores2d):
    info = plsc.get_sparse_core_info()
    nc, ns = info.num_cores, info.num_subcores
    nw = nc * ns
    rows_per_w = RH // nw

    @functools.partial(
        pl.kernel,
        out_type=jax.ShapeDtypeStruct((RH, 16), jnp.float32),
        mesh=plsc.VectorSubcoreMesh(core_axis_name="c", subcore_axis_name="s"),
        compiler_params=pltpu.CompilerParams(needs_layout_passes=False),
        scratch_types=[
            pltpu.VMEM((M_KEYS,), jnp.float32),
            pltpu.VMEM((M_KEYS,), jnp.float32),
            pltpu.VMEM((NBLK,), jnp.float32),
            pltpu.VMEM((NBLK,), jnp.float32),
            pltpu.VMEM((rows_per_w, 16), jnp.float32),
            pltpu.SemaphoreType.DMA,
            pltpu.SemaphoreType.DMA,
            pltpu.SemaphoreType.DMA,
            pltpu.SemaphoreType.DMA,
        ],
    )
    def k(bmax_hbm, s_hbm, out_hbm, srow0, srow1, bm0, bm1, o_v,
          sem_s0, sem_s1, sem_b0, sem_b1):
        wid = lax.axis_index("s") * nc + lax.axis_index("c")
        base = wid * rows_per_w
        lane = lax.iota(jnp.int32, 16)
        last = rows_per_w - 1
        srows = (srow0, srow1)
        bms = (bm0, bm1)
        sems_s = (sem_s0, sem_s1)
        sems_b = (sem_b0, sem_b1)

        for b in (0, 1):
            pltpu.async_copy(s_hbm.at[base + b], srows[b], sems_s[b])
            pltpu.async_copy(bmax_hbm.at[base + b], bms[b], sems_b[b])

        def process(i, buf):
            srow = srows[buf]
            bm = bms[buf]
            sem_s = sems_s[buf]
            sem_b = sems_b[buf]
            pltpu.make_async_copy(s_hbm.at[base], srow, sem_s).wait()
            pltpu.make_async_copy(bmax_hbm.at[base], bm, sem_b).wait()

            def load_bm(v):
                return bm[pl.ds(v * 16, 16)], lane + v * 16

            _, v1, _, v2 = _top32_of_refslab(load_bm, NBLK // 16)

            base1 = ((v1 & -128) << 4) | (v1 & 127)
            base2 = ((v2 & -128) << 4) | (v2 & 127)

            def load_cand(v):
                vb = base1 if v < 16 else base2
                return plsc.load_gather(srow, [vb + ((v % 16) << 7)])

            t32 = _rank32_of_keys(load_cand, DELTA)
            o_v[i] = jnp.full((16,), t32, jnp.float32)
            nxt = base + jnp.minimum(i + 2, last)
            pltpu.async_copy(s_hbm.at[nxt], srow, sem_s)
            pltpu.async_copy(bmax_hbm.at[nxt], bm, sem_b)

        def body(g, carry):
            process(2 * g, 0)
            process(2 * g + 1, 1)
            return carry

        lax.fori_loop(0, rows_per_w // 2, body, None)
        for b in (0, 1):
            pltpu.make_async_copy(s_hbm.at[base], srows[b], sems_s[b]).wait()
            pltpu.make_async_copy(bmax_hbm.at[base], bms[b], sems_b[b]).wait()
        pltpu.sync_copy(o_v, out_hbm.at[pl.ds(base, rows_per_w)])

    return k(bmax2d, scores2d)


def _combine_body(s_ref, t_ref, m_ref, o_ref, acc, zacc):
    j = pl.program_id(2)
    nj = pl.num_programs(2)

    @pl.when(j == 0)
    def _init():
        acc[...] = jnp.zeros_like(acc)
        zacc[...] = jnp.zeros_like(zacc)

    s = s_ref[0]
    t = t_ref[0][:, :1]
    alpha = jnp.where(s >= t, jnp.exp(TAU * s), 0.0)
    acc[...] += jax.lax.dot_general(
        alpha, m_ref[0], (((1,), (0,)), ((), ())), preferred_element_type=jnp.float32
    )
    zacc[...] += jnp.sum(alpha.reshape(BBLK, MBLK // 128, 128), axis=1)

    @pl.when(j == nj - 1)
    def _fin():
        z = jnp.sum(zacc[...], axis=1, keepdims=True)
        o_ref[...] = acc[...] / z


def _combine(scores, t32, M):
    grid = (QH, B // BBLK, M_KEYS // MBLK)
    return pl.pallas_call(
        _combine_body,
        grid=grid,
        in_specs=[
            pl.BlockSpec((1, BBLK, MBLK), lambda q, i, j: (q, i, j)),
            pl.BlockSpec((1, BBLK, 16), lambda q, i, j: (q, i, 0)),
            pl.BlockSpec((1, MBLK, U), lambda q, i, j: (q, j, 0)),
        ],
        out_specs=pl.BlockSpec((BBLK, U), lambda q, i, j: (i, q)),
        out_shape=jax.ShapeDtypeStruct((B, QH * U), jnp.float32),
        scratch_shapes=[
            pltpu.VMEM((BBLK, U), jnp.float32),
            pltpu.VMEM((BBLK, 128), jnp.float32),
        ],
    )(scores, t32, M)


def kernel(x, K, M):
    halves = []
    for h in range(2):
        ksl = K[h * QH:(h + 1) * QH]
        msl = M[h * QH:(h + 1) * QH]
        scores, bmax = _scores(x, ksl)
        top1632 = _t32_sc(bmax.reshape(RH, NBLK), scores.reshape(RH, M_KEYS))
        out = _combine(scores, top1632.reshape(QH, B, 16), msl)
        halves.append(out.reshape(B, QH, U))
    return jnp.concatenate(halves, axis=1)


# speedup vs baseline: 44.1880x; 1.0154x over previous
import functools
import math

import jax
import jax.numpy as jnp
from jax import lax
from jax.experimental import pallas as pl
from jax.experimental.pallas import tpu as pltpu
from jax.experimental.pallas import tpu_sc as plsc

Q = 8
D = 128
M_KEYS = 8192
U = 128
DELTA = 32
GAMMA_ALPHA = 0.1
B = 1024
TAU = GAMMA_ALPHA / math.sqrt(D)

R = Q * B
NSPLIT = 4
QH = Q // NSPLIT
RH = QH * B
NBLK = M_KEYS // 16

BBLK = 256
MBLK_A = 2048
MBLK = 2048


def _scores_body(x_ref, k_ref, s_ref, bm_ref):
    xb = x_ref[...]
    nrm = jnp.sqrt(jnp.sum(xb * xb, axis=1, keepdims=True))
    xb = xb / jnp.maximum(nrm, 1e-12)
    s = jax.lax.dot_general(
        xb, k_ref[0], (((1,), (1,)), ((), ())), preferred_element_type=jnp.float32
    )
    s_ref[0] = s
    bm_ref[0] = jnp.max(s.reshape(BBLK, 16, MBLK_A // 16), axis=1)


def _scores(x, K):
    grid = (QH, B // BBLK, M_KEYS // MBLK_A)
    return pl.pallas_call(
        _scores_body,
        grid=grid,
        in_specs=[
            pl.BlockSpec((BBLK, D), lambda q, i, j: (i, 0)),
            pl.BlockSpec((1, MBLK_A, D), lambda q, i, j: (q, j, 0)),
        ],
        out_specs=[
            pl.BlockSpec((1, BBLK, MBLK_A), lambda q, i, j: (q, i, j)),
            pl.BlockSpec((1, BBLK, MBLK_A // 16), lambda q, i, j: (q, i, j)),
        ],
        out_shape=[
            jax.ShapeDtypeStruct((QH, B, M_KEYS), jnp.float32),
            jax.ShapeDtypeStruct((QH, B, NBLK), jnp.float32),
        ],
    )(x, K)


def _rev(v):
    return lax.rev(v, (0,))


def _kv_sort(k, v):
    return plsc.sort_key_val(k, v, descending=True)


def _kv_minmax(ka, va, kb, vb):
    m = ka >= kb
    return (
        jnp.where(m, ka, kb),
        jnp.where(m, va, vb),
        jnp.where(m, kb, ka),
        jnp.where(m, vb, va),
    )


def _pair_to_sorted32(ka, va, kb, vb):
    kp, vp, kq, vq = _kv_minmax(ka, va, _rev(kb), _rev(vb))
    k1, v1 = _kv_sort(kp, vp)
    k2, v2 = _kv_sort(kq, vq)
    return k1, v1, k2, v2


def _merge32_top32(a, b):
    ka1, va1, ka2, va2 = a
    kb1, vb1, kb2, vb2 = b
    l1k, l1v, _, _ = _kv_minmax(ka1, va1, _rev(kb2), _rev(vb2))
    l2k, l2v, _, _ = _kv_minmax(ka2, va2, _rev(kb1), _rev(vb1))
    kp, vp, kq, vq = _kv_minmax(l1k, l1v, l2k, l2v)
    k1, v1 = _kv_sort(kp, vp)
    k2, v2 = _kv_sort(kq, vq)
    return (k1, v1, k2, v2)


def _top32_of_refslab(load_fn, nvec):
    leaves = []
    for v in range(0, nvec, 2):
        ka, va = load_fn(v)
        kb, vb = load_fn(v + 1)
        ka, va = _kv_sort(ka, va)
        kb, vb = _kv_sort(kb, vb)
        leaves.append(_pair_to_sorted32(ka, va, kb, vb))
    while len(leaves) > 1:
        nxt = []
        for i in range(0, len(leaves), 2):
            nxt.append(_merge32_top32(leaves[i], leaves[i + 1]))
        leaves = nxt
    return leaves[0]


def _k_sort(k):
    return plsc.sort_key_val(k, k, descending=True)[0]


def _keys_pair_to_sorted32(ka, kb):
    kp = jnp.maximum(ka, _rev(kb))
    kq = jnp.minimum(ka, _rev(kb))
    return _k_sort(kp), _k_sort(kq)


def _keys_merge32(a, b):
    ka1, ka2 = a
    kb1, kb2 = b
    l1 = jnp.maximum(ka1, _rev(kb2))
    l2 = jnp.maximum(ka2, _rev(kb1))
    kp = jnp.maximum(l1, l2)
    kq = jnp.minimum(l1, l2)
    return _k_sort(kp), _k_sort(kq)


def _rank32_of_keys(load_fn, nvec):
    leaves = []
    for v in range(0, nvec, 2):
        ka = _k_sort(load_fn(v))
        kb = _k_sort(load_fn(v + 1))
        leaves.append(_keys_pair_to_sorted32(ka, kb))
    while len(leaves) > 1:
        nxt = []
        for i in range(0, len(leaves), 2):
            nxt.append(_keys_merge32(leaves[i], leaves[i + 1]))
        leaves = nxt
    return lax.reduce_min(leaves[0][1], (0,))


def _t32_sc(bmax2d, scores2d):
    info = plsc.get_sparse_core_info()
    nc, ns = info.num_cores, info.num_subcores
    nw = nc * ns
    rows_per_w = RH // nw

    @functools.partial(
        pl.kernel,
        out_type=jax.ShapeDtypeStruct((RH, 16), jnp.float32),
        mesh=plsc.VectorSubcoreMesh(core_axis_name="c", subcore_axis_name="s"),
        compiler_params=pltpu.CompilerParams(needs_layout_passes=False),
        scratch_types=[
            pltpu.VMEM((M_KEYS,), jnp.float32),
            pltpu.VMEM((M_KEYS,), jnp.float32),
            pltpu.VMEM((NBLK,), jnp.float32),
            pltpu.VMEM((NBLK,), jnp.float32),
            pltpu.VMEM((rows_per_w, 16), jnp.float32),
            pltpu.SemaphoreType.DMA,
            pltpu.SemaphoreType.DMA,
            pltpu.SemaphoreType.DMA,
            pltpu.SemaphoreType.DMA,
        ],
    )
    def k(bmax_hbm, s_hbm, out_hbm, srow0, srow1, bm0, bm1, o_v,
          sem_s0, sem_s1, sem_b0, sem_b1):
        wid = lax.axis_index("s") * nc + lax.axis_index("c")
        base = wid * rows_per_w
        lane = lax.iota(jnp.int32, 16)
        last = rows_per_w - 1
        srows = (srow0, srow1)
        bms = (bm0, bm1)
        sems_s = (sem_s0, sem_s1)
        sems_b = (sem_b0, sem_b1)

        for b in (0, 1):
            pltpu.async_copy(s_hbm.at[base + b], srows[b], sems_s[b])
            pltpu.async_copy(bmax_hbm.at[base + b], bms[b], sems_b[b])

        def process(i, buf):
            srow = srows[buf]
            bm = bms[buf]
            sem_s = sems_s[buf]
            sem_b = sems_b[buf]
            pltpu.make_async_copy(s_hbm.at[base], srow, sem_s).wait()
            pltpu.make_async_copy(bmax_hbm.at[base], bm, sem_b).wait()

            def load_bm(v):
                return bm[pl.ds(v * 16, 16)], lane + v * 16

            _, v1, _, v2 = _top32_of_refslab(load_bm, NBLK // 16)

            base1 = ((v1 & -128) << 4) | (v1 & 127)
            base2 = ((v2 & -128) << 4) | (v2 & 127)

            def load_cand(v):
                vb = base1 if v < 16 else base2
                return plsc.load_gather(srow, [vb + ((v % 16) << 7)])

            t32 = _rank32_of_keys(load_cand, DELTA)
            o_v[i] = jnp.full((16,), t32, jnp.float32)
            nxt = base + jnp.minimum(i + 2, last)
            pltpu.async_copy(s_hbm.at[nxt], srow, sem_s)
            pltpu.async_copy(bmax_hbm.at[nxt], bm, sem_b)

        def body(g, carry):
            process(2 * g, 0)
            process(2 * g + 1, 1)
            return carry

        lax.fori_loop(0, rows_per_w // 2, body, None)
        for b in (0, 1):
            pltpu.make_async_copy(s_hbm.at[base], srows[b], sems_s[b]).wait()
            pltpu.make_async_copy(bmax_hbm.at[base], bms[b], sems_b[b]).wait()
        pltpu.sync_copy(o_v, out_hbm.at[pl.ds(base, rows_per_w)])

    return k(bmax2d, scores2d)


def _combine_body(s_ref, t_ref, m_ref, o_ref, acc, zacc):
    j = pl.program_id(2)
    nj = pl.num_programs(2)

    @pl.when(j == 0)
    def _init():
        acc[...] = jnp.zeros_like(acc)
        zacc[...] = jnp.zeros_like(zacc)

    s = s_ref[0]
    t = t_ref[0][:, :1]
    alpha = jnp.where(s >= t, jnp.exp(TAU * s), 0.0)
    acc[...] += jax.lax.dot_general(
        alpha, m_ref[0], (((1,), (0,)), ((), ())), preferred_element_type=jnp.float32
    )
    zacc[...] += jnp.sum(alpha.reshape(BBLK, MBLK // 128, 128), axis=1)

    @pl.when(j == nj - 1)
    def _fin():
        z = jnp.sum(zacc[...], axis=1, keepdims=True)
        o_ref[...] = acc[...] / z


def _combine(scores, t32, M):
    grid = (QH, B // BBLK, M_KEYS // MBLK)
    return pl.pallas_call(
        _combine_body,
        grid=grid,
        in_specs=[
            pl.BlockSpec((1, BBLK, MBLK), lambda q, i, j: (q, i, j)),
            pl.BlockSpec((1, BBLK, 16), lambda q, i, j: (q, i, 0)),
            pl.BlockSpec((1, MBLK, U), lambda q, i, j: (q, j, 0)),
        ],
        out_specs=pl.BlockSpec((BBLK, U), lambda q, i, j: (i, q)),
        out_shape=jax.ShapeDtypeStruct((B, QH * U), jnp.float32),
        scratch_shapes=[
            pltpu.VMEM((BBLK, U), jnp.float32),
            pltpu.VMEM((BBLK, 128), jnp.float32),
        ],
    )(scores, t32, M)


def kernel(x, K, M):
    halves = []
    for h in range(NSPLIT):
        ksl = K[h * QH:(h + 1) * QH]
        msl = M[h * QH:(h + 1) * QH]
        scores, bmax = _scores(x, ksl)
        top1632 = _t32_sc(bmax.reshape(RH, NBLK), scores.reshape(RH, M_KEYS))
        out = _combine(scores, top1632.reshape(QH, B, 16), msl)
        halves.append(out.reshape(B, QH, U))
    return jnp.concatenate(halves, axis=1)


# speedup vs baseline: 53.3415x; 1.2071x over previous
import functools
import math

import jax
import jax.numpy as jnp
from jax import lax
from jax.experimental import pallas as pl
from jax.experimental.pallas import tpu as pltpu
from jax.experimental.pallas import tpu_sc as plsc

Q = 8
D = 128
M_KEYS = 8192
U = 128
DELTA = 32
GAMMA_ALPHA = 0.1
B = 1024
TAU = GAMMA_ALPHA / math.sqrt(D)

R = Q * B
NSPLIT = 4
QH = Q // NSPLIT
RH = QH * B
NBLK = M_KEYS // 16

BBLK = 256
MBLK_A = 2048
MBLK = 2048


def _scores_body(x_ref, k_ref, s_ref, bm_ref):
    xb = x_ref[...]
    nrm = jnp.sqrt(jnp.sum(xb * xb, axis=1, keepdims=True))
    xb = xb / jnp.maximum(nrm, 1e-12)
    s = jax.lax.dot_general(
        xb, k_ref[0], (((1,), (1,)), ((), ())), preferred_element_type=jnp.float32
    )
    s_ref[0] = s
    bm_ref[0] = jnp.max(s.reshape(BBLK, 16, MBLK_A // 16), axis=1)


def _scores(x, K):
    grid = (QH, M_KEYS // MBLK_A, B // BBLK)
    return pl.pallas_call(
        _scores_body,
        grid=grid,
        in_specs=[
            pl.BlockSpec((BBLK, D), lambda q, j, i: (i, 0)),
            pl.BlockSpec((1, MBLK_A, D), lambda q, j, i: (q, j, 0)),
        ],
        out_specs=[
            pl.BlockSpec((1, BBLK, MBLK_A), lambda q, j, i: (q, i, j)),
            pl.BlockSpec((1, BBLK, MBLK_A // 16), lambda q, j, i: (q, i, j)),
        ],
        out_shape=[
            jax.ShapeDtypeStruct((QH, B, M_KEYS), jnp.float32),
            jax.ShapeDtypeStruct((QH, B, NBLK), jnp.float32),
        ],
    )(x, K)


def _rev(v):
    return lax.rev(v, (0,))


def _kv_sort(k, v):
    return plsc.sort_key_val(k, v, descending=True)


def _kv_minmax(ka, va, kb, vb):
    m = ka >= kb
    return (
        jnp.where(m, ka, kb),
        jnp.where(m, va, vb),
        jnp.where(m, kb, ka),
        jnp.where(m, vb, va),
    )


def _pair_to_sorted32(ka, va, kb, vb):
    kp, vp, kq, vq = _kv_minmax(ka, va, _rev(kb), _rev(vb))
    k1, v1 = _kv_sort(kp, vp)
    k2, v2 = _kv_sort(kq, vq)
    return k1, v1, k2, v2


def _merge32_top32(a, b):
    ka1, va1, ka2, va2 = a
    kb1, vb1, kb2, vb2 = b
    l1k, l1v, _, _ = _kv_minmax(ka1, va1, _rev(kb2), _rev(vb2))
    l2k, l2v, _, _ = _kv_minmax(ka2, va2, _rev(kb1), _rev(vb1))
    kp, vp, kq, vq = _kv_minmax(l1k, l1v, l2k, l2v)
    k1, v1 = _kv_sort(kp, vp)
    k2, v2 = _kv_sort(kq, vq)
    return (k1, v1, k2, v2)


def _top32_of_refslab(load_fn, nvec):
    leaves = []
    for v in range(0, nvec, 2):
        ka, va = load_fn(v)
        kb, vb = load_fn(v + 1)
        ka, va = _kv_sort(ka, va)
        kb, vb = _kv_sort(kb, vb)
        leaves.append(_pair_to_sorted32(ka, va, kb, vb))
    while len(leaves) > 1:
        nxt = []
        for i in range(0, len(leaves), 2):
            nxt.append(_merge32_top32(leaves[i], leaves[i + 1]))
        leaves = nxt
    return leaves[0]


def _k_sort(k):
    return plsc.sort_key_val(k, k, descending=True)[0]


def _keys_pair_to_sorted32(ka, kb):
    kp = jnp.maximum(ka, _rev(kb))
    kq = jnp.minimum(ka, _rev(kb))
    return _k_sort(kp), _k_sort(kq)


def _keys_merge32(a, b):
    ka1, ka2 = a
    kb1, kb2 = b
    l1 = jnp.maximum(ka1, _rev(kb2))
    l2 = jnp.maximum(ka2, _rev(kb1))
    kp = jnp.maximum(l1, l2)
    kq = jnp.minimum(l1, l2)
    return _k_sort(kp), _k_sort(kq)


def _rank32_of_keys(load_fn, nvec):
    leaves = []
    for v in range(0, nvec, 2):
        ka = _k_sort(load_fn(v))
        kb = _k_sort(load_fn(v + 1))
        leaves.append(_keys_pair_to_sorted32(ka, kb))
    while len(leaves) > 1:
        nxt = []
        for i in range(0, len(leaves), 2):
            nxt.append(_keys_merge32(leaves[i], leaves[i + 1]))
        leaves = nxt
    return lax.reduce_min(leaves[0][1], (0,))


def _t32_sc(bmax2d, scores2d):
    info = plsc.get_sparse_core_info()
    nc, ns = info.num_cores, info.num_subcores
    nw = nc * ns
    rows_per_w = RH // nw

    @functools.partial(
        pl.kernel,
        out_type=jax.ShapeDtypeStruct((RH, 16), jnp.float32),
        mesh=plsc.VectorSubcoreMesh(core_axis_name="c", subcore_axis_name="s"),
        compiler_params=pltpu.CompilerParams(needs_layout_passes=False),
        scratch_types=[
            pltpu.VMEM((M_KEYS,), jnp.float32),
            pltpu.VMEM((M_KEYS,), jnp.float32),
            pltpu.VMEM((NBLK,), jnp.float32),
            pltpu.VMEM((NBLK,), jnp.float32),
            pltpu.VMEM((rows_per_w, 16), jnp.float32),
            pltpu.SemaphoreType.DMA,
            pltpu.SemaphoreType.DMA,
            pltpu.SemaphoreType.DMA,
            pltpu.SemaphoreType.DMA,
        ],
    )
    def k(bmax_hbm, s_hbm, out_hbm, srow0, srow1, bm0, bm1, o_v,
          sem_s0, sem_s1, sem_b0, sem_b1):
        wid = lax.axis_index("s") * nc + lax.axis_index("c")
        base = wid * rows_per_w
        lane = lax.iota(jnp.int32, 16)
        last = rows_per_w - 1
        srows = (srow0, srow1)
        bms = (bm0, bm1)
        sems_s = (sem_s0, sem_s1)
        sems_b = (sem_b0, sem_b1)

        for b in (0, 1):
            pltpu.async_copy(s_hbm.at[base + b], srows[b], sems_s[b])
            pltpu.async_copy(bmax_hbm.at[base + b], bms[b], sems_b[b])

        def process(i, buf):
            srow = srows[buf]
            bm = bms[buf]
            sem_s = sems_s[buf]
            sem_b = sems_b[buf]
            pltpu.make_async_copy(s_hbm.at[base], srow, sem_s).wait()
            pltpu.make_async_copy(bmax_hbm.at[base], bm, sem_b).wait()

            def load_bm(v):
                return bm[pl.ds(v * 16, 16)], lane + v * 16

            _, v1, _, v2 = _top32_of_refslab(load_bm, NBLK // 16)

            base1 = ((v1 & -128) << 4) | (v1 & 127)
            base2 = ((v2 & -128) << 4) | (v2 & 127)

            def load_cand(v):
                vb = base1 if v < 16 else base2
                return plsc.load_gather(srow, [vb + ((v % 16) << 7)])

            t32 = _rank32_of_keys(load_cand, DELTA)
            o_v[i] = jnp.full((16,), t32, jnp.float32)
            nxt = base + jnp.minimum(i + 2, last)
            pltpu.async_copy(s_hbm.at[nxt], srow, sem_s)
            pltpu.async_copy(bmax_hbm.at[nxt], bm, sem_b)

        def body(g, carry):
            process(2 * g, 0)
            process(2 * g + 1, 1)
            return carry

        lax.fori_loop(0, rows_per_w // 2, body, None)
        for b in (0, 1):
            pltpu.make_async_copy(s_hbm.at[base], srows[b], sems_s[b]).wait()
            pltpu.make_async_copy(bmax_hbm.at[base], bms[b], sems_b[b]).wait()
        pltpu.sync_copy(o_v, out_hbm.at[pl.ds(base, rows_per_w)])

    return k(bmax2d, scores2d)


def _combine_body(s_ref, t_ref, m_ref, o_ref, acc, zacc):
    j = pl.program_id(1)
    nj = pl.num_programs(1)

    @pl.when(j == 0)
    def _init():
        acc[...] = jnp.zeros_like(acc)
        zacc[...] = jnp.zeros_like(zacc)

    s = s_ref[0]
    t = t_ref[0][:, :1]
    alpha = jnp.where(s >= t, jnp.exp(TAU * s), 0.0)
    acc[...] += jax.lax.dot_general(
        alpha, m_ref[0], (((1,), (0,)), ((), ())), preferred_element_type=jnp.float32
    )
    zacc[...] += jnp.sum(alpha.reshape(B, MBLK // 128, 128), axis=1)

    @pl.when(j == nj - 1)
    def _fin():
        z = jnp.sum(zacc[...], axis=1, keepdims=True)
        o_ref[...] = acc[...] / z


def _combine(scores, t32, M):
    grid = (QH, M_KEYS // MBLK)
    return pl.pallas_call(
        _combine_body,
        grid=grid,
        in_specs=[
            pl.BlockSpec((1, B, MBLK), lambda q, j: (q, 0, j)),
            pl.BlockSpec((1, B, 16), lambda q, j: (q, 0, 0)),
            pl.BlockSpec((1, MBLK, U), lambda q, j: (q, j, 0)),
        ],
        out_specs=pl.BlockSpec((B, U), lambda q, j: (0, q)),
        out_shape=jax.ShapeDtypeStruct((B, QH * U), jnp.float32),
        scratch_shapes=[
            pltpu.VMEM((B, U), jnp.float32),
            pltpu.VMEM((B, 128), jnp.float32),
        ],
    )(scores, t32, M)


def kernel(x, K, M):
    halves = []
    for h in range(NSPLIT):
        ksl = K[h * QH:(h + 1) * QH]
        msl = M[h * QH:(h + 1) * QH]
        scores, bmax = _scores(x, ksl)
        top1632 = _t32_sc(bmax.reshape(RH, NBLK), scores.reshape(RH, M_KEYS))
        out = _combine(scores, top1632.reshape(QH, B, 16), msl)
        halves.append(out.reshape(B, QH, U))
    return jnp.concatenate(halves, axis=1)
